# baseline (speedup 1.0000x reference)
import functools

import jax
import jax.numpy as jnp
from jax.experimental import pallas as pl

_T = 128


def _hnet_kernel(hs_cur_ref, hs_prev_ref, qw_ref, kw_ref, out_ref, qk_scr,
                 *, L, D):
    T = _T
    G = L // T
    f32 = jnp.float32
    b = pl.program_id(0)
    par = jax.lax.rem(b, 2)
    prv = jax.lax.rem(b + 1, 2)

    qw = qw_ref[...]
    kw = kw_ref[...]
    dn = (((1,), (1,)), ((), ()))
    hs_c = hs_cur_ref[0]
    qk_scr[par, :, :D] = jax.lax.dot_general(
        hs_c, qw, dn, preferred_element_type=f32)
    qk_scr[par, :, D:] = jax.lax.dot_general(
        hs_c, kw, dn, preferred_element_type=f32)

    hs = hs_prev_ref[0]
    q = qk_scr[prv, :, :D]
    k = qk_scr[prv, :, D:]

    k_next = jnp.concatenate([k[1:], jnp.zeros((1, D), f32)], axis=0)

    nq = jnp.maximum(jnp.sqrt(jnp.sum(q * q, axis=1, keepdims=True)), 1e-12)
    nk2c = jnp.sum(k_next * k_next, axis=1, keepdims=True)
    nk = jnp.maximum(jnp.sqrt(nk2c), 1e-12)
    dqk = jnp.sum(q * k_next, axis=1, keepdims=True)
    cos = dqk / (nq * nk)

    pm = jnp.clip((1.0 - cos) * 0.5, 0.0, 1.0)
    prob = jnp.concatenate([jnp.ones((1, 1), f32), pm[:L - 1]], axis=0)

    prob128 = jnp.broadcast_to(prob, (L, T))
    mask = prob128 > 0.5
    a = jnp.where(mask, 1.0 - prob128, 1.0)
    c = jnp.where(mask, prob128, 0.0)

    t_io = jax.lax.broadcasted_iota(jnp.int32, (G, T, T), 1)
    j_io = jax.lax.broadcasted_iota(jnp.int32, (G, T, T), 2)
    a3 = a.reshape(G, T, T)
    Lm3 = (jnp.where(t_io == j_io, 1.0, 0.0)
           + jnp.where(t_io == j_io + 1, a3, 0.0)).astype(f32)
    a3 = a3 * jnp.concatenate([jnp.ones((G, 1, T), f32), a3[:, :T - 1]], axis=1)
    s = 2
    while s < T:
        a_sh = jnp.concatenate([jnp.ones((G, s, T), f32), a3[:, :T - s]], axis=1)
        Lm_sh = jnp.concatenate([jnp.zeros((G, s, T), f32), Lm3[:, :T - s]], axis=1)
        Lm3 = Lm3 + a3 * Lm_sh
        a3 = a3 * a_sh
        s *= 2
    Lm = Lm3.reshape(L, T)
    a = a3.reshape(L, T)

    b_full = jnp.concatenate(
        [c * hs[:, j * T:(j + 1) * T] for j in range(D // T)], axis=1)

    s_locs = [
        jnp.dot(Lm[g * T:(g + 1) * T], b_full[g * T:(g + 1) * T],
                preferred_element_type=f32)
        for g in range(G)
    ]

    Sb = jnp.concatenate([sl[T - 1:T] for sl in s_locs], axis=0)
    Aa = jnp.concatenate(
        [a[g * T + T - 1:g * T + T] for g in range(G)], axis=0)
    s = 1
    while s < G:
        Sb_sh = jnp.concatenate([jnp.zeros((s, D), f32), Sb[:G - s]], axis=0)
        Aa_sh = jnp.concatenate([jnp.ones((s, T), f32), Aa[:G - s]], axis=0)
        Sb = Sb + jnp.concatenate(
            [Aa * Sb_sh[:, j * T:(j + 1) * T] for j in range(D // T)], axis=1)
        Aa = Aa * Aa_sh
        s *= 2
    S_prev = jnp.concatenate([jnp.zeros((1, D), f32), Sb[:G - 1]], axis=0)

    s_loc_full = jnp.concatenate(s_locs, axis=0)
    S_full = jnp.broadcast_to(S_prev.reshape(G, 1, D), (G, T, D)).reshape(L, D)
    corr = jnp.concatenate(
        [a * S_full[:, j * T:(j + 1) * T] for j in range(D // T)], axis=1)
    out_ref[0] = hs + s_loc_full + corr


def kernel(hidden_states, q_weight, k_weight):
    B, L, D = hidden_states.shape
    from jax.experimental.pallas import tpu as pltpu
    return pl.pallas_call(
        functools.partial(_hnet_kernel, L=L, D=D),
        grid=(B + 1,),
        in_specs=[
            pl.BlockSpec((1, L, D), lambda b: (jnp.minimum(b, B - 1), 0, 0)),
            pl.BlockSpec((1, L, D), lambda b: (jnp.maximum(b - 1, 0), 0, 0)),
            pl.BlockSpec((D, D), lambda b: (0, 0)),
            pl.BlockSpec((D, D), lambda b: (0, 0)),
        ],
        out_specs=pl.BlockSpec((1, L, D), lambda b: (jnp.maximum(b - 1, 0), 0, 0)),
        out_shape=jax.ShapeDtypeStruct((B, L, D), hidden_states.dtype),
        scratch_shapes=[pltpu.VMEM((2, L, 2 * D), jnp.float32)],
    )(hidden_states, hidden_states, q_weight, k_weight)


# speedup vs baseline: 1.4674x; 1.4674x over previous
import functools

import jax
import jax.numpy as jnp
from jax.experimental import pallas as pl

_T = 128


def _hnet_kernel(hs_ref, qw_ref, kw_ref, out_ref, *, L, D):
    T = _T
    G = L // T
    f32 = jnp.float32
    hs = hs_ref[0]
    qw = qw_ref[...]
    kw = kw_ref[...]

    dn = (((1,), (1,)), ((), ()))
    q = jax.lax.dot_general(hs, qw, dn, preferred_element_type=f32)
    k = jax.lax.dot_general(hs, kw, dn, preferred_element_type=f32)

    k_next = jnp.concatenate([k[1:], jnp.zeros((1, D), f32)], axis=0)

    nq = jnp.maximum(jnp.sqrt(jnp.sum(q * q, axis=1, keepdims=True)), 1e-12)
    nk2c = jnp.sum(k_next * k_next, axis=1, keepdims=True)
    nk = jnp.maximum(jnp.sqrt(nk2c), 1e-12)
    dqk = jnp.sum(q * k_next, axis=1, keepdims=True)
    cos = dqk / (nq * nk)

    pm = jnp.clip((1.0 - cos) * 0.5, 0.0, 1.0)
    prob = jnp.concatenate([jnp.ones((1, 1), f32), pm[:L - 1]], axis=0)

    prob128 = jnp.broadcast_to(prob, (L, T))
    mask = prob128 > 0.5
    a = jnp.where(mask, 1.0 - prob128, 1.0)
    c = jnp.where(mask, prob128, 0.0)

    t_io = jax.lax.broadcasted_iota(jnp.int32, (G, T, T), 1)
    j_io = jax.lax.broadcasted_iota(jnp.int32, (G, T, T), 2)
    a3 = a.reshape(G, T, T)
    Lm3 = (jnp.where(t_io == j_io, 1.0, 0.0)
           + jnp.where(t_io == j_io + 1, a3, 0.0)).astype(f32)
    a3 = a3 * jnp.concatenate([jnp.ones((G, 1, T), f32), a3[:, :T - 1]], axis=1)
    s = 2
    while s < T:
        a_sh = jnp.concatenate([jnp.ones((G, s, T), f32), a3[:, :T - s]], axis=1)
        Lm_sh = jnp.concatenate([jnp.zeros((G, s, T), f32), Lm3[:, :T - s]], axis=1)
        Lm3 = Lm3 + a3 * Lm_sh
        a3 = a3 * a_sh
        s *= 2
    Lm = Lm3.reshape(L, T)
    a = a3.reshape(L, T)

    b_full = jnp.concatenate(
        [c * hs[:, j * T:(j + 1) * T] for j in range(D // T)], axis=1)

    s_locs = [
        jnp.dot(Lm[g * T:(g + 1) * T], b_full[g * T:(g + 1) * T],
                preferred_element_type=f32)
        for g in range(G)
    ]

    Sb = jnp.concatenate([sl[T - 1:T] for sl in s_locs], axis=0)
    Aa = jnp.concatenate(
        [a[g * T + T - 1:g * T + T] for g in range(G)], axis=0)
    s = 1
    while s < G:
        Sb_sh = jnp.concatenate([jnp.zeros((s, D), f32), Sb[:G - s]], axis=0)
        Aa_sh = jnp.concatenate([jnp.ones((s, T), f32), Aa[:G - s]], axis=0)
        Sb = Sb + jnp.concatenate(
            [Aa * Sb_sh[:, j * T:(j + 1) * T] for j in range(D // T)], axis=1)
        Aa = Aa * Aa_sh
        s *= 2
    S_prev = jnp.concatenate([jnp.zeros((1, D), f32), Sb[:G - 1]], axis=0)

    s_loc_full = jnp.concatenate(s_locs, axis=0)
    S_full = jnp.broadcast_to(S_prev.reshape(G, 1, D), (G, T, D)).reshape(L, D)
    corr = jnp.concatenate(
        [a * S_full[:, j * T:(j + 1) * T] for j in range(D // T)], axis=1)
    out_ref[0] = hs + s_loc_full + corr


def kernel(hidden_states, q_weight, k_weight):
    B, L, D = hidden_states.shape
    return pl.pallas_call(
        functools.partial(_hnet_kernel, L=L, D=D),
        grid=(B,),
        in_specs=[
            pl.BlockSpec((1, L, D), lambda b: (b, 0, 0)),
            pl.BlockSpec((D, D), lambda b: (0, 0)),
            pl.BlockSpec((D, D), lambda b: (0, 0)),
        ],
        out_specs=pl.BlockSpec((1, L, D), lambda b: (b, 0, 0)),
        out_shape=jax.ShapeDtypeStruct((B, L, D), hidden_states.dtype),
    )(hidden_states, q_weight, k_weight)
